# Initial kernel scaffold; baseline (speedup 1.0000x reference)
import jax
import jax.numpy as jnp
from jax.experimental import pallas as pl


def kernel(edge_index, features, W1, b1, W2, b2):
    raise NotImplementedError("write your pallas kernel here")


# speedup vs baseline: 22.0321x; 22.0321x over previous
import jax
import jax.numpy as jnp
from jax import lax
from jax.experimental import pallas as pl
from jax.experimental.pallas import tpu as pltpu
from jax.experimental.pallas import tpu_sc as plsc

N = 10000
F = 128
E = 320000
NP = 10240
NC, NS, L = 2, 16, 16
NW = NC * NS
CH = 128
EPT = 10240
EP = NW * EPT
NCHUNK = EPT // CH
PACK = 16384
STRIPE = NP // NS
NEG = 5 * N
POS_CHUNKS = NCHUNK
NEG_CHUNKS = 16
LCHUNKS = POS_CHUNKS + NEG_CHUNKS
NEGP = NW * NEG_CHUNKS * CH

_mesh = lambda: plsc.VectorSubcoreMesh(
    core_axis_name="c", subcore_axis_name="s", num_cores=NC, num_subcores=NS)


def _sc_deg_body(colc_hbm, out_hbm, idx_v, ones_v, z_v, hist_sh):
    cid = lax.axis_index("c")
    sid = lax.axis_index("s")
    wid = cid * NS + sid
    pltpu.sync_copy(colc_hbm.at[pl.ds(wid * NCHUNK, NCHUNK)], idx_v)
    zeros16 = jnp.zeros((L,), jnp.float32)
    ones16 = jnp.ones((L,), jnp.float32)

    def zb(i, c):
        z_v[pl.ds(i * L, L)] = zeros16
        return c
    lax.fori_loop(0, STRIPE // L, zb, 0)

    def ob(i, c):
        ones_v[pl.ds(i * L, L)] = ones16
        return c
    lax.fori_loop(0, CH // L, ob, 0)

    pltpu.sync_copy(z_v, hist_sh.at[pl.ds(sid * STRIPE, STRIPE)])
    plsc.subcore_barrier()

    def sc(j, c):
        pltpu.sync_copy(ones_v, hist_sh.at[idx_v.at[j]], add=True)
        return c
    lax.fori_loop(0, NCHUNK, sc, 0)
    plsc.subcore_barrier()
    pltpu.sync_copy(hist_sh.at[pl.ds(sid * STRIPE, STRIPE)],
                    out_hbm.at[cid, pl.ds(sid * STRIPE, STRIPE)])


_sc_deg = pl.kernel(
    _sc_deg_body,
    out_type=jax.ShapeDtypeStruct((NC, NP), jnp.float32),
    mesh=_mesh(),
    scratch_types=[
        pltpu.VMEM((NCHUNK, CH), jnp.int32),
        pltpu.VMEM((CH,), jnp.float32),
        pltpu.VMEM((STRIPE,), jnp.float32),
        pltpu.VMEM_SHARED((NP,), jnp.float32),
    ],
)


def _sc_push_body(su_hbm, pc_hbm, out_hbm,
                  pidx_v, ridx0_v, ridx1_v, cidx0_v, cidx1_v,
                  rows_v, acc_sh, semA, semB):
    cid = lax.axis_index("c")
    sid = lax.axis_index("s")
    wid = cid * NS + sid
    pltpu.sync_copy(pc_hbm.at[pl.ds(wid * NCHUNK, NCHUNK)], pidx_v)
    pltpu.sync_copy(su_hbm.at[pl.ds(sid * STRIPE, STRIPE)],
                    acc_sh.at[pl.ds(sid * STRIPE, STRIPE)])
    plsc.subcore_barrier()

    def unpack(j, ridx, cidx):
        for g in range(CH // L):
            v = pidx_v[j, pl.ds(g * L, L)]
            ridx[pl.ds(g * L, L)] = lax.shift_right_logical(v, 14)
            cidx[pl.ds(g * L, L)] = lax.bitwise_and(v, PACK - 1)

    def start(ridx, b, sem):
        pltpu.async_copy(su_hbm.at[ridx], rows_v.at[b], sem)

    def wait(ridx, b, sem):
        pltpu.make_async_copy(su_hbm.at[ridx], rows_v.at[b], sem).wait()

    unpack(0, ridx0_v, cidx0_v)
    start(ridx0_v, 0, semA)

    def body(k, c):
        jA = 2 * k
        jB = jA + 1
        unpack(jB, ridx1_v, cidx1_v)
        start(ridx1_v, 1, semB)
        wait(ridx0_v, 0, semA)
        pltpu.sync_copy(rows_v.at[0], acc_sh.at[cidx0_v], add=True)

        @pl.when(k < NCHUNK // 2 - 1)
        def _():
            unpack(jB + 1, ridx0_v, cidx0_v)
            start(ridx0_v, 0, semA)

        wait(ridx1_v, 1, semB)
        pltpu.sync_copy(rows_v.at[1], acc_sh.at[cidx1_v], add=True)
        return c
    lax.fori_loop(0, NCHUNK // 2, body, 0)
    plsc.subcore_barrier()
    pltpu.sync_copy(acc_sh.at[pl.ds(sid * STRIPE, STRIPE)],
                    out_hbm.at[cid, pl.ds(sid * STRIPE, STRIPE)])


_sc_push = pl.kernel(
    _sc_push_body,
    out_type=jax.ShapeDtypeStruct((NC, NP, F), jnp.float32),
    mesh=_mesh(),
    scratch_types=[
        pltpu.VMEM((NCHUNK, CH), jnp.int32),
        pltpu.VMEM((CH,), jnp.int32),
        pltpu.VMEM((CH,), jnp.int32),
        pltpu.VMEM((CH,), jnp.int32),
        pltpu.VMEM((CH,), jnp.int32),
        pltpu.VMEM((2, CH, F), jnp.float32),
        pltpu.VMEM_SHARED((NP, F), jnp.float32),
        pltpu.SemaphoreType.DMA,
        pltpu.SemaphoreType.DMA,
    ],
)


def _sc_loss_body(rep_hbm, p0c_hbm, p1c_hbm, out_hbm,
                  idx0_v, idx1_v, rows0_v, rows1_v, acc_v,
                  s0A, s0B, s1A, s1B):
    cid = lax.axis_index("c")
    sid = lax.axis_index("s")
    wid = cid * NS + sid
    base = wid * LCHUNKS
    pltpu.sync_copy(p0c_hbm.at[pl.ds(base, LCHUNKS)], idx0_v)
    pltpu.sync_copy(p1c_hbm.at[pl.ds(base, LCHUNKS)], idx1_v)
    lanes = lax.iota(jnp.int32, L)

    def start(j, b):
        pltpu.async_copy(rep_hbm.at[idx0_v.at[j]], rows0_v.at[b],
                         s0A if b == 0 else s0B)
        pltpu.async_copy(rep_hbm.at[idx1_v.at[j]], rows1_v.at[b],
                         s1A if b == 0 else s1B)

    def wait(j, b):
        pltpu.make_async_copy(rep_hbm.at[idx0_v.at[j]], rows0_v.at[b],
                              s0A if b == 0 else s0B).wait()
        pltpu.make_async_copy(rep_hbm.at[idx1_v.at[j]], rows1_v.at[b],
                              s1A if b == 0 else s1B).wait()

    perms = [(lanes + k) % L for k in (8, 4, 2, 1)]

    def hsum(p):
        for pm in perms:
            p = p + jnp.take(p, pm)
        return p

    def chunk(j, b, carry):
        loss_a, cnt_a = carry
        tgt_s = jnp.where(j < POS_CHUNKS, 1.0, 0.0)
        tgt = jnp.full((L,), tgt_s, jnp.float32)
        rb0 = rows0_v.at[b]
        rb1 = rows1_v.at[b]
        for g in range(CH // L):
            i0 = idx0_v[j, pl.ds(g * L, L)]
            i1 = idx1_v[j, pl.ds(g * L, L)]
            mf = jnp.where(i0 < i1, 1.0, 0.0)

            def ebody(u, dv):
                e = g * L + u
                p = jnp.zeros((L,), jnp.float32)
                for k in range(F // L):
                    p = p + rb0[e, pl.ds(k * L, L)] * rb1[e, pl.ds(k * L, L)]
                s = hsum(p)
                return jnp.where(lanes == u, s, dv)
            dot = lax.fori_loop(0, L, ebody, jnp.zeros((L,), jnp.float32))
            d = dot - tgt
            loss_a = loss_a + d * d * mf
            cnt_a = cnt_a + mf
        return loss_a, cnt_a

    start(0, 0)

    def body(k, carry):
        jA = 2 * k
        jB = jA + 1
        start(jB, 1)
        wait(jA, 0)
        carry = chunk(jA, 0, carry)

        @pl.when(k < LCHUNKS // 2 - 1)
        def _():
            start(jB + 1, 0)

        wait(jB, 1)
        carry = chunk(jB, 1, carry)
        return carry

    z = jnp.zeros((L,), jnp.float32)
    loss_a, cnt_a = lax.fori_loop(0, LCHUNKS // 2, body, (z, z))
    acc_v[0, :] = loss_a
    acc_v[1, :] = cnt_a
    pltpu.sync_copy(acc_v, out_hbm.at[wid])


_sc_loss = pl.kernel(
    _sc_loss_body,
    out_type=jax.ShapeDtypeStruct((NW, 2, L), jnp.float32),
    mesh=_mesh(),
    scratch_types=[
        pltpu.VMEM((LCHUNKS, CH), jnp.int32),
        pltpu.VMEM((LCHUNKS, CH), jnp.int32),
        pltpu.VMEM((2, CH, F), jnp.float32),
        pltpu.VMEM((2, CH, F), jnp.float32),
        pltpu.VMEM((2, L), jnp.float32),
        pltpu.SemaphoreType.DMA,
        pltpu.SemaphoreType.DMA,
        pltpu.SemaphoreType.DMA,
        pltpu.SemaphoreType.DMA,
    ],
)


BLK = 1024


def _tc1_body(x_ref, w_ref, d0_ref, d1_ref, o_ref):
    dinv = lax.rsqrt(d0_ref[...] + d1_ref[...] + 1.0)
    o_ref[...] = jnp.dot(x_ref[...], w_ref[...],
                         preferred_element_type=jnp.float32) * dinv


_tc1 = pl.pallas_call(
    _tc1_body,
    grid=(NP // BLK,),
    in_specs=[
        pl.BlockSpec((BLK, F), lambda i: (i, 0)),
        pl.BlockSpec((F, F), lambda i: (0, 0)),
        pl.BlockSpec((BLK, 1), lambda i: (i, 0)),
        pl.BlockSpec((BLK, 1), lambda i: (i, 0)),
    ],
    out_specs=pl.BlockSpec((BLK, F), lambda i: (i, 0)),
    out_shape=jax.ShapeDtypeStruct((NP, F), jnp.float32),
)


def _tc23_body(a0_ref, a1_ref, su_ref, d0_ref, d1_ref, b1_ref, b2_ref,
               w_ref, flag_ref, o_ref):
    dinv = lax.rsqrt(d0_ref[...] + d1_ref[...] + 1.0)
    base = (a0_ref[...] + a1_ref[...] - su_ref[...]) * dinv
    h = jnp.maximum(base + b1_ref[...], 0.0)
    o1 = jnp.dot(h, w_ref[...], preferred_element_type=jnp.float32) * dinv
    o2 = base + b2_ref[...]
    o_ref[...] = jnp.where(flag_ref[0, 0] > 0.5, o1, o2)


_tc23 = pl.pallas_call(
    _tc23_body,
    grid=(NP // BLK,),
    in_specs=[
        pl.BlockSpec((BLK, F), lambda i: (i, 0)),
        pl.BlockSpec((BLK, F), lambda i: (i, 0)),
        pl.BlockSpec((BLK, F), lambda i: (i, 0)),
        pl.BlockSpec((BLK, 1), lambda i: (i, 0)),
        pl.BlockSpec((BLK, 1), lambda i: (i, 0)),
        pl.BlockSpec((1, F), lambda i: (0, 0)),
        pl.BlockSpec((1, F), lambda i: (0, 0)),
        pl.BlockSpec((F, F), lambda i: (0, 0)),
        pl.BlockSpec((1, 1), lambda i: (0, 0)),
    ],
    out_specs=pl.BlockSpec((BLK, F), lambda i: (i, 0)),
    out_shape=jax.ShapeDtypeStruct((NP, F), jnp.float32),
)


def kernel(edge_index, features, W1, b1, W2, b2):
    ei = edge_index
    ar = jnp.arange(EP - E, dtype=jnp.int32)
    row_p = jnp.concatenate([ei[0], (ar * 37) % N])
    col_p = jnp.concatenate([ei[1], N + (ar % (NP - N))])
    pc = (row_p * PACK + col_p).reshape(EP // CH, CH)
    colc_deg = col_p.reshape(EP // CH, CH)
    x_p = jnp.pad(features, ((0, NP - N), (0, 0)))

    deg_parts = _sc_deg(colc_deg)
    d0 = deg_parts[0].reshape(NP, 1)
    d1 = deg_parts[1].reshape(NP, 1)

    su1 = _tc1(x_p, W1, d0, d1)
    b1r = b1.reshape(1, F)
    b2r = b2.reshape(1, F)

    def _layer(carry, _):
        step, su = carry
        accs = _sc_push(su, pc)
        flag = (step < 1).astype(jnp.float32).reshape(1, 1)
        out = _tc23(accs[0], accs[1], su, d0, d1, b1r, b2r, W2, flag)
        return (step + 1, out), None

    (_, rep_p), _ = lax.scan(_layer, (jnp.int32(0), su1), None, length=2)

    neg = jax.random.randint(jax.random.key(42), (2, NEG), 0, N,
                             dtype=jnp.int32)
    a2 = jnp.arange(EP - E, dtype=jnp.int32)
    pos0 = jnp.concatenate([ei[0], (N // 2) + (a2 % (N // 2))])
    pos1 = jnp.concatenate([ei[1], a2 % (N // 2)])
    a3 = jnp.arange(NEGP - NEG, dtype=jnp.int32)
    neg0 = jnp.concatenate([neg[0], (N // 2) + (a3 % (N // 2))])
    neg1 = jnp.concatenate([neg[1], a3 % (N // 2)])
    p0 = jnp.concatenate([pos0.reshape(NW, POS_CHUNKS, CH),
                          neg0.reshape(NW, NEG_CHUNKS, CH)],
                         axis=1).reshape(NW * LCHUNKS, CH)
    p1 = jnp.concatenate([pos1.reshape(NW, POS_CHUNKS, CH),
                          neg1.reshape(NW, NEG_CHUNKS, CH)],
                         axis=1).reshape(NW * LCHUNKS, CH)

    parts = _sc_loss(rep_p, p0, p1)
    loss_sum = jnp.sum(parts[:, 0, :])
    cnt = jnp.sum(parts[:, 1, :])
    rec_loss = loss_sum * N / cnt
    return rep_p[:N], rec_loss
